# baseline (speedup 1.0000x reference)
import functools

import jax
import jax.numpy as jnp
from jax import lax
from jax.experimental import pallas as pl
from jax.experimental.pallas import tpu as pltpu
from jax.experimental.pallas import tpu_sc as plsc

N = 10000
D = 128
E = 320000
NPAD = 10240
NC = 2
NS = 16
NW = NC * NS
EPT = E // NW
CH = 128
NCH = -(-EPT // CH)
EPT_PAD = NCH * CH
PSH = 14
ACC = 10016
ROWS0 = 632
ROWS_LAST = ACC - (NS - 1) * ROWS0
BLK = 512
GRID = NPAD // BLK

_SC_MESH = plsc.VectorSubcoreMesh(core_axis_name="c", subcore_axis_name="s")
_SC_PARAMS = pltpu.CompilerParams(needs_layout_passes=False)


def _deg_body(r_hbm, c_hbm, degp_hbm, rc_hbm, rbuf, cbuf, cobuf, deg_v):
    wid = lax.axis_index("c") * NS + lax.axis_index("s")
    base = wid * EPT
    pltpu.sync_copy(r_hbm.at[pl.ds(base, EPT)], rbuf)
    pltpu.sync_copy(c_hbm.at[pl.ds(base, EPT)], cbuf)

    def zero(i, carry):
        deg_v[pl.ds(i * 16, 16)] = jnp.zeros((16,), jnp.float32)
        return carry

    lax.fori_loop(0, NPAD // 16, zero, 0)

    def step(i, carry):
        r = rbuf[pl.ds(i * 16, 16)]
        c = cbuf[pl.ds(i * 16, 16)]
        m = r != c
        plsc.addupdate_scatter(deg_v, [r], jnp.ones((16,), jnp.float32), mask=m)
        cobuf[pl.ds(i * 16, 16)] = lax.bitwise_or(
            r, jnp.where(m, c, N) << PSH)
        return carry

    lax.fori_loop(0, EPT // 16, step, 0)
    sink = N + lax.iota(jnp.int32, 16)
    padv = lax.bitwise_or(sink, sink << PSH)
    for t in range(EPT, EPT_PAD, 16):
        cobuf[pl.ds(t, 16)] = padv
    pltpu.sync_copy(deg_v, degp_hbm.at[wid])
    pltpu.sync_copy(cobuf, rc_hbm.at[wid])


_deg_kernel = pl.kernel(
    _deg_body,
    out_type=[
        jax.ShapeDtypeStruct((NW, NPAD), jnp.float32),
        jax.ShapeDtypeStruct((NW, EPT_PAD), jnp.int32),
    ],
    mesh=_SC_MESH,
    scratch_types=[
        pltpu.VMEM((EPT,), jnp.int32),
        pltpu.VMEM((EPT,), jnp.int32),
        pltpu.VMEM((EPT_PAD,), jnp.int32),
        pltpu.VMEM((NPAD,), jnp.float32),
    ],
    compiler_params=_SC_PARAMS,
)


NBUF = 3
SLACK = 1
NI = 2 * NBUF


def _prop_body(xp_hbm, rc_hbm, out_hbm,
               ibuf, cidx_v, rows_v, acc_sh, gsem, ssem, isem):
    core = lax.axis_index("c")
    sid = lax.axis_index("s")
    wid = core * NS + sid

    def fetch_idx(k):
        pltpu.async_copy(rc_hbm.at[wid, k], ibuf.at[lax.rem(k, NI)], isem)

    def unpack(j, b):
        s = lax.rem(j, NI)
        for v in range(CH // 16):
            rc = ibuf[s, pl.ds(v * 16, 16)]
            cidx_v[b, pl.ds(v * 16, 16)] = lax.shift_right_logical(rc, PSH)
            ibuf[s, pl.ds(v * 16, 16)] = lax.bitwise_and(rc, (1 << PSH) - 1)

    def zb(i, carry):
        for j in range(D // 16):
            rows_v[0, i, pl.ds(j * 16, 16)] = jnp.zeros((16,), jnp.float32)
        return carry

    lax.fori_loop(0, CH, zb, 0)

    def zero_rows(base, nrows):
        full, rem = divmod(nrows, CH)
        for k in range(full):
            pltpu.sync_copy(rows_v.at[0], acc_sh.at[pl.ds(base + k * CH, CH)])
        if rem:
            pltpu.sync_copy(rows_v.at[0, pl.ds(0, rem)],
                            acc_sh.at[pl.ds(base + full * CH, rem)])

    @pl.when(sid < NS - 1)
    def _():
        zero_rows(sid * ROWS0, ROWS0)

    @pl.when(sid == NS - 1)
    def _():
        zero_rows((NS - 1) * ROWS0, ROWS_LAST)

    plsc.subcore_barrier()

    def wait_one(sem):
        pltpu.make_async_copy(xp_hbm.at[pl.ds(0, CH)], rows_v.at[0], sem).wait()

    def wait_idx():
        pltpu.make_async_copy(rc_hbm.at[0, 0], ibuf.at[0], isem).wait()

    for k in range(NI):
        fetch_idx(k)
    for b in range(NBUF):
        wait_idx()
        unpack(b, b)
        pltpu.async_copy(xp_hbm.at[ibuf.at[b]], rows_v.at[b], gsem)

    def chunk(i, carry):
        wait_one(gsem)

        @pl.when((i >= SLACK) & (i - SLACK + NBUF < NCH))
        def _():
            wait_one(ssem)
            j = i - SLACK + NBUF
            b = lax.rem(j, NBUF)
            wait_idx()
            unpack(j, b)

            @pl.when(i - SLACK + 2 * NBUF < NCH)
            def _():
                fetch_idx(i - SLACK + 2 * NBUF)

            pltpu.async_copy(xp_hbm.at[ibuf.at[lax.rem(j, NI)]],
                             rows_v.at[b], gsem)

        pltpu.async_copy(rows_v.at[lax.rem(i, NBUF)],
                         acc_sh.at[cidx_v.at[lax.rem(i, NBUF)]], ssem, add=True)
        return carry

    lax.fori_loop(0, NCH, chunk, 0)
    for _ in range(NBUF):
        wait_one(ssem)
    plsc.subcore_barrier()

    @pl.when(sid < NS - 1)
    def _():
        pltpu.sync_copy(acc_sh.at[pl.ds(sid * ROWS0, ROWS0)],
                        out_hbm.at[core, pl.ds(sid * ROWS0, ROWS0)])

    @pl.when(sid == NS - 1)
    def _():
        pltpu.sync_copy(acc_sh.at[pl.ds((NS - 1) * ROWS0, ROWS_LAST)],
                        out_hbm.at[core, pl.ds((NS - 1) * ROWS0, ROWS_LAST)])


_prop_kernel = pl.kernel(
    _prop_body,
    out_type=jax.ShapeDtypeStruct((NC, NPAD, D), jnp.float32),
    mesh=_SC_MESH,
    scratch_types=[
        pltpu.VMEM((NI, CH), jnp.int32),
        pltpu.VMEM((NBUF, CH), jnp.int32),
        pltpu.VMEM((NBUF, CH, D), jnp.float32),
        pltpu.VMEM_SHARED((ACC, D), jnp.float32),
        pltpu.SemaphoreType.DMA,
        pltpu.SemaphoreType.DMA,
        pltpu.SemaphoreType.DMA,
    ],
    compiler_params=_SC_PARAMS,
)


def _prep_body(degp_ref, x_ref, dis_ref, dsc_ref, xp_ref):
    i = pl.program_id(0)
    row = lax.broadcasted_iota(jnp.int32, (BLK, 1), 0) + i * BLK
    deg = jnp.sum(degp_ref[...], axis=1, keepdims=True)
    deg = deg + jnp.where(row < N, 1.0, 0.0)
    good = deg > 0.0
    safe = jnp.maximum(deg, 1.0)
    dis = jnp.where(good, lax.rsqrt(safe), 0.0)
    dsc = jnp.where(good, 2.0 / safe, 0.0)
    dis_ref[...] = jnp.broadcast_to(dis, (BLK, D))
    dsc_ref[...] = jnp.broadcast_to(dsc, (BLK, D))
    xp_ref[...] = dis * x_ref[...]


_prep_kernel = pl.pallas_call(
    _prep_body,
    grid=(GRID,),
    in_specs=[
        pl.BlockSpec((BLK, NW), lambda i: (i, 0)),
        pl.BlockSpec((BLK, D), lambda i: (i, 0)),
    ],
    out_specs=[
        pl.BlockSpec((BLK, D), lambda i: (i, 0)),
        pl.BlockSpec((BLK, D), lambda i: (i, 0)),
        pl.BlockSpec((BLK, D), lambda i: (i, 0)),
    ],
    out_shape=[
        jax.ShapeDtypeStruct((NPAD, D), jnp.float32),
        jax.ShapeDtypeStruct((NPAD, D), jnp.float32),
        jax.ShapeDtypeStruct((NPAD, D), jnp.float32),
    ],
)


def _layer_body(acc_ref, x_ref, dis_ref, dsc_ref, w_ref, h_ref, xp_ref, *, relu):
    x = x_ref[...]
    sup = dis_ref[...] * (acc_ref[0] + acc_ref[1]) + dsc_ref[...] * x
    out = (jnp.dot(sup, w_ref[0:D, :], preferred_element_type=jnp.float32)
           + jnp.dot(x, w_ref[D:2 * D, :], preferred_element_type=jnp.float32))
    mu = jnp.mean(out, axis=-1, keepdims=True)
    ctr = out - mu
    var = jnp.mean(ctr * ctr, axis=-1, keepdims=True)
    y = ctr * lax.rsqrt(var + 1e-5)
    if relu:
        y = jnp.maximum(y, 0.0)
    h_ref[...] = y
    xp_ref[...] = dis_ref[...] * y


def _make_layer_kernel(relu):
    return pl.pallas_call(
        functools.partial(_layer_body, relu=relu),
        grid=(GRID,),
        in_specs=[
            pl.BlockSpec((NC, BLK, D), lambda i: (0, i, 0)),
            pl.BlockSpec((BLK, D), lambda i: (i, 0)),
            pl.BlockSpec((BLK, D), lambda i: (i, 0)),
            pl.BlockSpec((BLK, D), lambda i: (i, 0)),
            pl.BlockSpec((2 * D, D), lambda i: (0, 0)),
        ],
        out_specs=[
            pl.BlockSpec((BLK, D), lambda i: (i, 0)),
            pl.BlockSpec((BLK, D), lambda i: (i, 0)),
        ],
        out_shape=[
            jax.ShapeDtypeStruct((NPAD, D), jnp.float32),
            jax.ShapeDtypeStruct((NPAD, D), jnp.float32),
        ],
    )


_layer_relu = _make_layer_kernel(True)
_layer_plain = _make_layer_kernel(False)


def kernel(x, edge_index, W0, W1, W2):
    ridx = edge_index[0].astype(jnp.int32)
    cidx = edge_index[1].astype(jnp.int32)

    degp, rc_flat = _deg_kernel(ridx, cidx)
    rc = rc_flat.reshape(NW, NCH, CH)

    x_pad = jnp.pad(x, ((0, NPAD - N), (0, 0)))
    dis128, dsc128, xp = _prep_kernel(degp.T, x_pad)

    h = x_pad
    for W, layer_fn in ((W0, _layer_relu), (W1, _layer_relu), (W2, _layer_plain)):
        acc = _prop_kernel(xp, rc)
        h, xp = layer_fn(acc, h, dis128, dsc128, W)
    return h[:N]


# speedup vs baseline: 1.0587x; 1.0587x over previous
import functools

import jax
import jax.numpy as jnp
from jax import lax
from jax.experimental import pallas as pl
from jax.experimental.pallas import tpu as pltpu
from jax.experimental.pallas import tpu_sc as plsc

N = 10000
D = 128
E = 320000
NPAD = 10240
NC = 2
NS = 16
NW = NC * NS
EPT = E // NW
CH = 80
NCH = -(-EPT // CH)
EPT_PAD = NCH * CH
PSH = 14
ACC = 10016
ROWS0 = 632
ROWS_LAST = ACC - (NS - 1) * ROWS0
BLK = 512
GRID = NPAD // BLK

_SC_MESH = plsc.VectorSubcoreMesh(core_axis_name="c", subcore_axis_name="s")
_SC_PARAMS = pltpu.CompilerParams(needs_layout_passes=False)


def _deg_body(r_hbm, c_hbm, degp_hbm, rc_hbm, rbuf, cbuf, cobuf, deg_v):
    wid = lax.axis_index("c") * NS + lax.axis_index("s")
    base = wid * EPT
    pltpu.sync_copy(r_hbm.at[pl.ds(base, EPT)], rbuf)
    pltpu.sync_copy(c_hbm.at[pl.ds(base, EPT)], cbuf)

    def zero(i, carry):
        deg_v[pl.ds(i * 16, 16)] = jnp.zeros((16,), jnp.float32)
        return carry

    lax.fori_loop(0, NPAD // 16, zero, 0)

    def step(i, carry):
        r = rbuf[pl.ds(i * 16, 16)]
        c = cbuf[pl.ds(i * 16, 16)]
        m = r != c
        plsc.addupdate_scatter(deg_v, [r], jnp.ones((16,), jnp.float32), mask=m)
        cobuf[pl.ds(i * 16, 16)] = lax.bitwise_or(
            r, jnp.where(m, c, N) << PSH)
        return carry

    lax.fori_loop(0, EPT // 16, step, 0)
    sink = N + lax.iota(jnp.int32, 16)
    padv = lax.bitwise_or(sink, sink << PSH)
    for t in range(EPT, EPT_PAD, 16):
        cobuf[pl.ds(t, 16)] = padv
    pltpu.sync_copy(deg_v, degp_hbm.at[wid])
    pltpu.sync_copy(cobuf, rc_hbm.at[wid])


_deg_kernel = pl.kernel(
    _deg_body,
    out_type=[
        jax.ShapeDtypeStruct((NW, NPAD), jnp.float32),
        jax.ShapeDtypeStruct((NW, EPT_PAD), jnp.int32),
    ],
    mesh=_SC_MESH,
    scratch_types=[
        pltpu.VMEM((EPT,), jnp.int32),
        pltpu.VMEM((EPT,), jnp.int32),
        pltpu.VMEM((EPT_PAD,), jnp.int32),
        pltpu.VMEM((NPAD,), jnp.float32),
    ],
    compiler_params=_SC_PARAMS,
)


NBUF = 4
SLACK = 1
NI = 2 * NBUF


def _prop_body(xp_hbm, rc_hbm, out_hbm,
               ibuf, cidx_v, rows_v, acc_sh, gsem, ssem, isem):
    core = lax.axis_index("c")
    sid = lax.axis_index("s")
    wid = core * NS + sid

    def fetch_idx(k):
        pltpu.async_copy(rc_hbm.at[wid, k], ibuf.at[lax.rem(k, NI)], isem)

    def unpack(j, b):
        s = lax.rem(j, NI)
        for v in range(CH // 16):
            rc = ibuf[s, pl.ds(v * 16, 16)]
            cidx_v[b, pl.ds(v * 16, 16)] = lax.shift_right_logical(rc, PSH)
            ibuf[s, pl.ds(v * 16, 16)] = lax.bitwise_and(rc, (1 << PSH) - 1)

    def zb(i, carry):
        for j in range(D // 16):
            rows_v[0, i, pl.ds(j * 16, 16)] = jnp.zeros((16,), jnp.float32)
        return carry

    lax.fori_loop(0, CH, zb, 0)

    def zero_rows(base, nrows):
        full, rem = divmod(nrows, CH)
        for k in range(full):
            pltpu.sync_copy(rows_v.at[0], acc_sh.at[pl.ds(base + k * CH, CH)])
        if rem:
            pltpu.sync_copy(rows_v.at[0, pl.ds(0, rem)],
                            acc_sh.at[pl.ds(base + full * CH, rem)])

    @pl.when(sid < NS - 1)
    def _():
        zero_rows(sid * ROWS0, ROWS0)

    @pl.when(sid == NS - 1)
    def _():
        zero_rows((NS - 1) * ROWS0, ROWS_LAST)

    plsc.subcore_barrier()

    def wait_one(sem):
        pltpu.make_async_copy(xp_hbm.at[pl.ds(0, CH)], rows_v.at[0], sem).wait()

    def wait_idx():
        pltpu.make_async_copy(rc_hbm.at[0, 0], ibuf.at[0], isem).wait()

    for k in range(NI):
        fetch_idx(k)
    for b in range(NBUF):
        wait_idx()
        unpack(b, b)
        pltpu.async_copy(xp_hbm.at[ibuf.at[b]], rows_v.at[b], gsem)

    def chunk(i, carry):
        wait_one(gsem)

        @pl.when((i >= SLACK) & (i - SLACK + NBUF < NCH))
        def _():
            wait_one(ssem)
            j = i - SLACK + NBUF
            b = lax.rem(j, NBUF)
            wait_idx()
            unpack(j, b)

            @pl.when(i - SLACK + 2 * NBUF < NCH)
            def _():
                fetch_idx(i - SLACK + 2 * NBUF)

            pltpu.async_copy(xp_hbm.at[ibuf.at[lax.rem(j, NI)]],
                             rows_v.at[b], gsem)

        pltpu.async_copy(rows_v.at[lax.rem(i, NBUF)],
                         acc_sh.at[cidx_v.at[lax.rem(i, NBUF)]], ssem, add=True)
        return carry

    lax.fori_loop(0, NCH, chunk, 0)
    for _ in range(NBUF):
        wait_one(ssem)
    plsc.subcore_barrier()

    @pl.when(sid < NS - 1)
    def _():
        pltpu.sync_copy(acc_sh.at[pl.ds(sid * ROWS0, ROWS0)],
                        out_hbm.at[core, pl.ds(sid * ROWS0, ROWS0)])

    @pl.when(sid == NS - 1)
    def _():
        pltpu.sync_copy(acc_sh.at[pl.ds((NS - 1) * ROWS0, ROWS_LAST)],
                        out_hbm.at[core, pl.ds((NS - 1) * ROWS0, ROWS_LAST)])


_prop_kernel = pl.kernel(
    _prop_body,
    out_type=jax.ShapeDtypeStruct((NC, NPAD, D), jnp.float32),
    mesh=_SC_MESH,
    scratch_types=[
        pltpu.VMEM((NI, CH), jnp.int32),
        pltpu.VMEM((NBUF, CH), jnp.int32),
        pltpu.VMEM((NBUF, CH, D), jnp.float32),
        pltpu.VMEM_SHARED((ACC, D), jnp.float32),
        pltpu.SemaphoreType.DMA,
        pltpu.SemaphoreType.DMA,
        pltpu.SemaphoreType.DMA,
    ],
    compiler_params=_SC_PARAMS,
)


def _prep_body(degp_ref, x_ref, dis_ref, dsc_ref, xp_ref):
    i = pl.program_id(0)
    row = lax.broadcasted_iota(jnp.int32, (BLK, 1), 0) + i * BLK
    deg = jnp.sum(degp_ref[...], axis=1, keepdims=True)
    deg = deg + jnp.where(row < N, 1.0, 0.0)
    good = deg > 0.0
    safe = jnp.maximum(deg, 1.0)
    dis = jnp.where(good, lax.rsqrt(safe), 0.0)
    dsc = jnp.where(good, 2.0 / safe, 0.0)
    dis_ref[...] = jnp.broadcast_to(dis, (BLK, D))
    dsc_ref[...] = jnp.broadcast_to(dsc, (BLK, D))
    xp_ref[...] = dis * x_ref[...]


_prep_kernel = pl.pallas_call(
    _prep_body,
    grid=(GRID,),
    in_specs=[
        pl.BlockSpec((BLK, NW), lambda i: (i, 0)),
        pl.BlockSpec((BLK, D), lambda i: (i, 0)),
    ],
    out_specs=[
        pl.BlockSpec((BLK, D), lambda i: (i, 0)),
        pl.BlockSpec((BLK, D), lambda i: (i, 0)),
        pl.BlockSpec((BLK, D), lambda i: (i, 0)),
    ],
    out_shape=[
        jax.ShapeDtypeStruct((NPAD, D), jnp.float32),
        jax.ShapeDtypeStruct((NPAD, D), jnp.float32),
        jax.ShapeDtypeStruct((NPAD, D), jnp.float32),
    ],
)


def _layer_body(acc_ref, x_ref, dis_ref, dsc_ref, w_ref, h_ref, xp_ref, *, relu):
    x = x_ref[...]
    sup = dis_ref[...] * (acc_ref[0] + acc_ref[1]) + dsc_ref[...] * x
    out = (jnp.dot(sup, w_ref[0:D, :], preferred_element_type=jnp.float32)
           + jnp.dot(x, w_ref[D:2 * D, :], preferred_element_type=jnp.float32))
    mu = jnp.mean(out, axis=-1, keepdims=True)
    ctr = out - mu
    var = jnp.mean(ctr * ctr, axis=-1, keepdims=True)
    y = ctr * lax.rsqrt(var + 1e-5)
    if relu:
        y = jnp.maximum(y, 0.0)
    h_ref[...] = y
    xp_ref[...] = dis_ref[...] * y


def _make_layer_kernel(relu):
    return pl.pallas_call(
        functools.partial(_layer_body, relu=relu),
        grid=(GRID,),
        in_specs=[
            pl.BlockSpec((NC, BLK, D), lambda i: (0, i, 0)),
            pl.BlockSpec((BLK, D), lambda i: (i, 0)),
            pl.BlockSpec((BLK, D), lambda i: (i, 0)),
            pl.BlockSpec((BLK, D), lambda i: (i, 0)),
            pl.BlockSpec((2 * D, D), lambda i: (0, 0)),
        ],
        out_specs=[
            pl.BlockSpec((BLK, D), lambda i: (i, 0)),
            pl.BlockSpec((BLK, D), lambda i: (i, 0)),
        ],
        out_shape=[
            jax.ShapeDtypeStruct((NPAD, D), jnp.float32),
            jax.ShapeDtypeStruct((NPAD, D), jnp.float32),
        ],
    )


_layer_relu = _make_layer_kernel(True)
_layer_plain = _make_layer_kernel(False)


def kernel(x, edge_index, W0, W1, W2):
    ridx = edge_index[0].astype(jnp.int32)
    cidx = edge_index[1].astype(jnp.int32)

    degp, rc_flat = _deg_kernel(ridx, cidx)
    rc = rc_flat.reshape(NW, NCH, CH)

    x_pad = jnp.pad(x, ((0, NPAD - N), (0, 0)))
    dis128, dsc128, xp = _prep_kernel(degp.T, x_pad)

    h = x_pad
    for W, layer_fn in ((W0, _layer_relu), (W1, _layer_relu), (W2, _layer_plain)):
        acc = _prop_kernel(xp, rc)
        h, xp = layer_fn(acc, h, dis128, dsc128, W)
    return h[:N]
